# baseline (device time: 56000 ns/iter reference)
import jax
import jax.numpy as jnp
from jax import lax
from jax.experimental import pallas as pl
from jax.experimental.pallas import tpu as pltpu

N_DEV = 4
HEADS = 8
GROUP = 4
DH = 128
SCALE = 0.08838834764831843


def kernel(x, Wq, Wo, Wk, Wv):
    B, Sq, D = x.shape
    dq = Wq.shape[1]

    def body(x_ref, wq_ref, wo_ref, wk_ref, wv_ref, out_ref,
             attn_ref, comm_ref, send_sems, recv_sems):
        my_pos = lax.axis_index("i")
        left = (my_pos + N_DEV - 1) % N_DEV
        right = (my_pos + 1) % N_DEV

        barrier_sem = pltpu.get_barrier_semaphore()
        for nbr in (left, right):
            pl.semaphore_signal(barrier_sem, inc=1, device_id=(nbr,),
                                device_id_type=pl.DeviceIdType.MESH)
        pl.semaphore_wait(barrier_sem, 2)

        x2d = x_ref[0, :, :]
        q = jnp.dot(x2d, wq_ref[...], preferred_element_type=jnp.float32)

        kv_cols = pl.ds(my_pos * 2 * DH, 2 * DH)
        k = jnp.dot(x2d, wk_ref[:, kv_cols], preferred_element_type=jnp.float32)
        v = jnp.dot(x2d, wv_ref[:, kv_cols], preferred_element_type=jnp.float32)

        for h in range(HEADS):
            qh = q[:, h * DH:(h + 1) * DH]
            g = h // GROUP
            kh = k[:, g * DH:(g + 1) * DH]
            vh = v[:, g * DH:(g + 1) * DH]
            s = lax.dot_general(qh, kh, (((1,), (1,)), ((), ())),
                                preferred_element_type=jnp.float32) * SCALE
            m = jnp.max(s, axis=1, keepdims=True)
            p = jnp.exp(s - m)
            l = jnp.sum(p, axis=1, keepdims=True)
            o = jnp.dot(p, vh, preferred_element_type=jnp.float32) / l
            attn_ref[:, h * DH:(h + 1) * DH] = o

        comm_ref[0, :, :] = jnp.dot(attn_ref[:, :], wo_ref[:, :],
                                    preferred_element_type=jnp.float32)

        for h in range(N_DEV - 1):
            rdma = pltpu.make_async_remote_copy(
                src_ref=comm_ref.at[h],
                dst_ref=comm_ref.at[h + 1],
                send_sem=send_sems.at[h],
                recv_sem=recv_sems.at[h],
                device_id=(right,),
                device_id_type=pl.DeviceIdType.MESH,
            )
            rdma.start()
            rdma.wait()

        out_ref[0, :, :] = (
            (comm_ref[0, :, :] + comm_ref[1, :, :])
            + (comm_ref[2, :, :] + comm_ref[3, :, :])
        )

    return pl.pallas_call(
        body,
        out_shape=jax.ShapeDtypeStruct((B, Sq, D), jnp.float32),
        in_specs=[pl.BlockSpec(memory_space=pltpu.VMEM)] * 5,
        out_specs=pl.BlockSpec(memory_space=pltpu.VMEM),
        scratch_shapes=[
            pltpu.VMEM((Sq, dq), jnp.float32),
            pltpu.VMEM((N_DEV, Sq, D), jnp.float32),
            pltpu.SemaphoreType.DMA((N_DEV - 1,)),
            pltpu.SemaphoreType.DMA((N_DEV - 1,)),
        ],
        compiler_params=pltpu.CompilerParams(collective_id=0),
    )(x, Wq, Wo, Wk, Wv)


# device time: 32121 ns/iter; 1.7434x vs baseline; 1.7434x over previous
import jax
import jax.numpy as jnp
from jax import lax
from jax.experimental import pallas as pl
from jax.experimental.pallas import tpu as pltpu

N_DEV = 4
HEADS = 8
GROUP = 4
DH = 128
SCALE = 0.08838834764831843


def kernel(x, Wq, Wo, Wk, Wv):
    B, Sq, D = x.shape
    dq = Wq.shape[1]
    H = D // 2
    Qt = D // 4
    E = D // 8

    def body(x_ref, wq_ref, wo_ref, wk_ref, wv_ref, out_ref,
             attn_ref, acc_ref, sa1_ref, sb1_ref, sa2_ref, sb2_ref,
             send_sems, recv_sems):
        my_pos = lax.axis_index("i")
        u = (my_pos % 2) ^ (my_pos // 2)
        v = my_pos // 2
        p1 = my_pos ^ 1
        p2 = 3 - my_pos

        barrier_sem = pltpu.get_barrier_semaphore()
        for nbr in (p1, p2):
            pl.semaphore_signal(barrier_sem, inc=1, device_id=(nbr,),
                                device_id_type=pl.DeviceIdType.MESH)
        pl.semaphore_wait(barrier_sem, 2)

        x2d = x_ref[0, :, :]
        q = jnp.dot(x2d, wq_ref[...], preferred_element_type=jnp.float32)

        kv_cols = pl.ds(my_pos * 2 * DH, 2 * DH)
        k = jnp.dot(x2d, wk_ref[:, kv_cols], preferred_element_type=jnp.float32)
        vv = jnp.dot(x2d, wv_ref[:, kv_cols], preferred_element_type=jnp.float32)

        for h in range(HEADS):
            qh = q[:, h * DH:(h + 1) * DH]
            g = h // GROUP
            kh = k[:, g * DH:(g + 1) * DH]
            vh = vv[:, g * DH:(g + 1) * DH]
            s = lax.dot_general(qh, kh, (((1,), (1,)), ((), ())),
                                preferred_element_type=jnp.float32) * SCALE
            m = jnp.max(s, axis=1, keepdims=True)
            p = jnp.exp(s - m)
            l = jnp.sum(p, axis=1, keepdims=True)
            o = jnp.dot(p, vh, preferred_element_type=jnp.float32) / l
            attn_ref[:, h * DH:(h + 1) * DH] = o

        acc_ref[:, :] = jnp.dot(attn_ref[:, :], wo_ref[:, :],
                                preferred_element_type=jnp.float32)

        a_keep = u * Qt
        a_send = (1 - u) * Qt
        aq_keep = u * Qt + v * E
        aq_send = u * Qt + (1 - v) * E
        b_keep = H + v * Qt
        b_send = H + (1 - v) * Qt
        bq_keep = H + v * Qt + u * E
        bq_send = H + v * Qt + (1 - u) * E

        def xchg(sem_idx, src_cols, w, dst_ref, partner):
            return pltpu.make_async_remote_copy(
                src_ref=acc_ref.at[:, pl.ds(src_cols, w)],
                dst_ref=dst_ref,
                send_sem=send_sems.at[sem_idx],
                recv_sem=recv_sems.at[sem_idx],
                device_id=(partner,),
                device_id_type=pl.DeviceIdType.MESH,
            )

        ra = xchg(0, a_send, Qt, sa1_ref, p1)
        rb = xchg(1, b_send, Qt, sb1_ref, p2)
        ra.start()
        rb.start()
        ra.wait()
        rb.wait()
        acc_ref[:, pl.ds(a_keep, Qt)] += sa1_ref[:, :]
        acc_ref[:, pl.ds(b_keep, Qt)] += sb1_ref[:, :]

        ra = xchg(2, aq_send, E, sa2_ref, p2)
        rb = xchg(3, bq_send, E, sb2_ref, p1)
        ra.start()
        rb.start()
        ra.wait()
        rb.wait()
        acc_ref[:, pl.ds(aq_keep, E)] += sa2_ref[:, :]
        acc_ref[:, pl.ds(bq_keep, E)] += sb2_ref[:, :]

        ra = xchg(4, aq_keep, E, acc_ref.at[:, pl.ds(aq_keep, E)], p2)
        rb = xchg(5, bq_keep, E, acc_ref.at[:, pl.ds(bq_keep, E)], p1)
        ra.start()
        rb.start()
        ra.wait()
        rb.wait()

        ra = xchg(6, a_keep, Qt, acc_ref.at[:, pl.ds(a_keep, Qt)], p1)
        rb = xchg(7, b_keep, Qt, acc_ref.at[:, pl.ds(b_keep, Qt)], p2)
        ra.start()
        rb.start()
        ra.wait()
        rb.wait()

        out_ref[0, :, :] = acc_ref[:, :]

    return pl.pallas_call(
        body,
        out_shape=jax.ShapeDtypeStruct((B, Sq, D), jnp.float32),
        in_specs=[pl.BlockSpec(memory_space=pltpu.VMEM)] * 5,
        out_specs=pl.BlockSpec(memory_space=pltpu.VMEM),
        scratch_shapes=[
            pltpu.VMEM((Sq, dq), jnp.float32),
            pltpu.VMEM((Sq, D), jnp.float32),
            pltpu.VMEM((Sq, Qt), jnp.float32),
            pltpu.VMEM((Sq, Qt), jnp.float32),
            pltpu.VMEM((Sq, E), jnp.float32),
            pltpu.VMEM((Sq, E), jnp.float32),
            pltpu.SemaphoreType.DMA((8,)),
            pltpu.SemaphoreType.DMA((8,)),
        ],
        compiler_params=pltpu.CompilerParams(collective_id=0),
    )(x, Wq, Wo, Wk, Wv)


# device time: 31922 ns/iter; 1.7543x vs baseline; 1.0062x over previous
import jax
import jax.numpy as jnp
from jax import lax
from jax.experimental import pallas as pl
from jax.experimental.pallas import tpu as pltpu

N_DEV = 4
HEADS = 8
GROUP = 4
DH = 128
SCALE = 0.08838834764831843


def kernel(x, Wq, Wo, Wk, Wv):
    B, Sq, D = x.shape
    dq = Wq.shape[1]
    H = D // 2
    Qt = D // 4
    E = D // 8

    def body(x_ref, wq_ref, wo_ref, wk_ref, wv_ref, out_ref,
             attn_ref, acc_ref, sa1_ref, sb1_ref, sa2_ref, sb2_ref,
             send_sems, recv_sems):
        my_pos = lax.axis_index("i")
        u = (my_pos % 2) ^ (my_pos // 2)
        v = my_pos // 2
        p1 = my_pos ^ 1
        p2 = 3 - my_pos

        barrier_sem = pltpu.get_barrier_semaphore()
        for nbr in (p1, p2):
            pl.semaphore_signal(barrier_sem, inc=1, device_id=(nbr,),
                                device_id_type=pl.DeviceIdType.MESH)
        pl.semaphore_wait(barrier_sem, 2)

        x2d = x_ref[0, :, :].astype(jnp.bfloat16)
        q = jnp.dot(x2d, wq_ref[...].astype(jnp.bfloat16),
                    preferred_element_type=jnp.float32)

        kv_cols = pl.ds(my_pos * 2 * DH, 2 * DH)
        k = jnp.dot(x2d, wk_ref[:, kv_cols].astype(jnp.bfloat16),
                    preferred_element_type=jnp.float32).astype(jnp.bfloat16)
        vv = jnp.dot(x2d, wv_ref[:, kv_cols].astype(jnp.bfloat16),
                     preferred_element_type=jnp.float32).astype(jnp.bfloat16)

        qb = q.astype(jnp.bfloat16)
        for h in range(HEADS):
            qh = qb[:, h * DH:(h + 1) * DH]
            g = h // GROUP
            kh = k[:, g * DH:(g + 1) * DH]
            vh = vv[:, g * DH:(g + 1) * DH]
            s = lax.dot_general(qh, kh, (((1,), (1,)), ((), ())),
                                preferred_element_type=jnp.float32) * SCALE
            m = jnp.max(s, axis=1, keepdims=True)
            p = jnp.exp(s - m)
            l = jnp.sum(p, axis=1, keepdims=True)
            o = jnp.dot(p.astype(jnp.bfloat16), vh,
                        preferred_element_type=jnp.float32) / l
            attn_ref[:, h * DH:(h + 1) * DH] = o.astype(jnp.bfloat16)

        a_keep = u * Qt
        a_send = (1 - u) * Qt
        aq_keep = u * Qt + v * E
        aq_send = u * Qt + (1 - v) * E
        b_keep = H + v * Qt
        b_send = H + (1 - v) * Qt
        bq_keep = H + v * Qt + u * E
        bq_send = H + v * Qt + (1 - u) * E

        def xchg(sem_idx, src_cols, w, dst_ref, partner):
            return pltpu.make_async_remote_copy(
                src_ref=acc_ref.at[:, pl.ds(src_cols, w)],
                dst_ref=dst_ref,
                send_sem=send_sems.at[sem_idx],
                recv_sem=recv_sems.at[sem_idx],
                device_id=(partner,),
                device_id_type=pl.DeviceIdType.MESH,
            )

        attn_b = attn_ref[:, :]
        for cs in (a_send, b_send):
            acc_ref[:, pl.ds(cs, Qt)] = jnp.dot(
                attn_b, wo_ref[:, pl.ds(cs, Qt)].astype(jnp.bfloat16),
                preferred_element_type=jnp.float32)

        ra = xchg(0, a_send, Qt, sa1_ref, p1)
        rb = xchg(1, b_send, Qt, sb1_ref, p2)
        ra.start()
        rb.start()

        for cs in (a_keep, b_keep):
            acc_ref[:, pl.ds(cs, Qt)] = jnp.dot(
                attn_b, wo_ref[:, pl.ds(cs, Qt)].astype(jnp.bfloat16),
                preferred_element_type=jnp.float32)

        ra.wait()
        rb.wait()
        acc_ref[:, pl.ds(a_keep, Qt)] += sa1_ref[:, :]
        acc_ref[:, pl.ds(b_keep, Qt)] += sb1_ref[:, :]

        ra = xchg(2, aq_send, E, sa2_ref, p2)
        rb = xchg(3, bq_send, E, sb2_ref, p1)
        ra.start()
        rb.start()
        ra.wait()
        rb.wait()
        acc_ref[:, pl.ds(aq_keep, E)] += sa2_ref[:, :]
        acc_ref[:, pl.ds(bq_keep, E)] += sb2_ref[:, :]

        ra = xchg(4, aq_keep, E, acc_ref.at[:, pl.ds(aq_keep, E)], p2)
        rb = xchg(5, bq_keep, E, acc_ref.at[:, pl.ds(bq_keep, E)], p1)
        ra.start()
        rb.start()
        ra.wait()
        rb.wait()

        ra = xchg(6, a_keep, Qt, acc_ref.at[:, pl.ds(a_keep, Qt)], p1)
        rb = xchg(7, b_keep, Qt, acc_ref.at[:, pl.ds(b_keep, Qt)], p2)
        ra.start()
        rb.start()
        ra.wait()
        rb.wait()

        out_ref[0, :, :] = acc_ref[:, :]

    return pl.pallas_call(
        body,
        out_shape=jax.ShapeDtypeStruct((B, Sq, D), jnp.float32),
        in_specs=[pl.BlockSpec(memory_space=pltpu.VMEM)] * 5,
        out_specs=pl.BlockSpec(memory_space=pltpu.VMEM),
        scratch_shapes=[
            pltpu.VMEM((Sq, dq), jnp.bfloat16),
            pltpu.VMEM((Sq, D), jnp.float32),
            pltpu.VMEM((Sq, Qt), jnp.float32),
            pltpu.VMEM((Sq, Qt), jnp.float32),
            pltpu.VMEM((Sq, E), jnp.float32),
            pltpu.VMEM((Sq, E), jnp.float32),
            pltpu.SemaphoreType.DMA((8,)),
            pltpu.SemaphoreType.DMA((8,)),
        ],
        compiler_params=pltpu.CompilerParams(collective_id=0),
    )(x, Wq, Wo, Wk, Wv)


# device time: 24357 ns/iter; 2.2991x vs baseline; 1.3106x over previous
import jax
import jax.numpy as jnp
from jax import lax
from jax.experimental import pallas as pl
from jax.experimental.pallas import tpu as pltpu

N_DEV = 4
HEADS = 8
GROUP = 4
DH = 128
KVW = 2 * DH
SCALE = 0.08838834764831843


def kernel(x, Wq, Wo, Wk, Wv):
    B, Sq, D = x.shape
    dq = Wq.shape[1]
    Qt = D // 4

    def body(x_ref, wq_ref, wo_ref, wk_ref, wv_ref, out_ref,
             acc_ref, r1_ref, r2_ref, kw_ref, vw_ref, ow_ref,
             send_sems, recv_sems, copy_sems):
        my_pos = lax.axis_index("i")
        p1 = my_pos ^ 1
        p2 = 3 - my_pos

        barrier_sem = pltpu.get_barrier_semaphore()
        for nbr in (p1, p2):
            pl.semaphore_signal(barrier_sem, inc=1, device_id=(nbr,),
                                device_id_type=pl.DeviceIdType.MESH)

        kv_cols = pl.ds(my_pos * KVW, KVW)
        ck = pltpu.make_async_copy(wk_ref.at[:, kv_cols], kw_ref,
                                   copy_sems.at[0])
        cv = pltpu.make_async_copy(wv_ref.at[:, kv_cols], vw_ref,
                                   copy_sems.at[1])
        ck.start()
        cv.start()
        co = pltpu.make_async_copy(wo_ref, ow_ref, copy_sems.at[2])
        co.start()

        x2d = x_ref[0, :, :].astype(jnp.bfloat16)
        q = jnp.dot(x2d, wq_ref[...].astype(jnp.bfloat16),
                    preferred_element_type=jnp.float32)

        ck.wait()
        cv.wait()
        k = jnp.dot(x2d, kw_ref[:, :].astype(jnp.bfloat16),
                    preferred_element_type=jnp.float32).astype(jnp.bfloat16)
        vv = jnp.dot(x2d, vw_ref[:, :].astype(jnp.bfloat16),
                     preferred_element_type=jnp.float32).astype(jnp.bfloat16)

        qb = q.astype(jnp.bfloat16)

        quarters = [slice(i * Qt, (i + 1) * Qt) for i in range(4)]
        r1_partner = [p1, p1, p2, p2]
        r2_partner = [p2, p2, p1, p1]
        RH = Sq // 2
        row_halves = [slice(0, RH), slice(RH, Sq)]

        def xchg(sem_idx, rows, cols, dst_ref, partner):
            return pltpu.make_async_remote_copy(
                src_ref=acc_ref.at[rows, cols],
                dst_ref=dst_ref,
                send_sem=send_sems.at[sem_idx],
                recv_sem=recv_sems.at[sem_idx],
                device_id=(partner,),
                device_id_type=pl.DeviceIdType.MESH,
            )

        r1 = []
        for rb, rows in enumerate(row_halves):
            o_parts = []
            for g in range(2):
                qsg = jnp.concatenate(
                    [qb[rows, (GROUP * g + j) * DH:(GROUP * g + j + 1) * DH]
                     for j in range(GROUP)], axis=0)
                kh = k[:, g * DH:(g + 1) * DH]
                vh = vv[:, g * DH:(g + 1) * DH]
                s = lax.dot_general(qsg, kh, (((1,), (1,)), ((), ())),
                                    preferred_element_type=jnp.float32) * SCALE
                m = jnp.max(s, axis=1, keepdims=True)
                p = jnp.exp(s - m)
                l = jnp.sum(p, axis=1, keepdims=True)
                o = jnp.dot(p.astype(jnp.bfloat16), vh,
                            preferred_element_type=jnp.float32) / l
                ob = o.astype(jnp.bfloat16)
                o_parts.extend(ob[j * RH:(j + 1) * RH, :]
                               for j in range(GROUP))
            attn_rb = jnp.concatenate(o_parts, axis=1)

            if rb == 0:
                co.wait()
            for i, qs in enumerate(quarters):
                acc_ref[rows, qs] = jnp.dot(
                    attn_rb, ow_ref[:, qs].astype(jnp.bfloat16),
                    preferred_element_type=jnp.float32).astype(jnp.bfloat16)
                if rb == 0 and i == 0:
                    pl.semaphore_wait(barrier_sem, 2)
                r = xchg(rb * 4 + i, rows, qs, r1_ref.at[rows, qs],
                         r1_partner[i])
                r.start()
                r1.append(r)

        chunks = [(rows, qs) for rows in row_halves for qs in quarters]
        r2 = []
        for idx, (rows, qs) in enumerate(chunks):
            r1[idx].wait()
            acc_ref[rows, qs] += r1_ref[rows, qs]
            r = xchg(8 + idx, rows, qs, r2_ref.at[rows, qs],
                     r2_partner[idx % 4])
            r.start()
            r2.append(r)

        for idx, (rows, qs) in enumerate(chunks):
            r2[idx].wait()
            out_ref[0, rows, qs] = (acc_ref[rows, qs].astype(jnp.float32)
                                    + r2_ref[rows, qs].astype(jnp.float32))

    return pl.pallas_call(
        body,
        out_shape=jax.ShapeDtypeStruct((B, Sq, D), jnp.float32),
        in_specs=[pl.BlockSpec(memory_space=pltpu.VMEM)] * 2
        + [pl.BlockSpec(memory_space=pl.ANY)] * 3,
        out_specs=pl.BlockSpec(memory_space=pltpu.VMEM),
        scratch_shapes=[
            pltpu.VMEM((Sq, D), jnp.bfloat16),
            pltpu.VMEM((Sq, D), jnp.bfloat16),
            pltpu.VMEM((Sq, D), jnp.bfloat16),
            pltpu.VMEM((D, KVW), jnp.float32),
            pltpu.VMEM((D, KVW), jnp.float32),
            pltpu.VMEM((dq, D), jnp.float32),
            pltpu.SemaphoreType.DMA((16,)),
            pltpu.SemaphoreType.DMA((16,)),
            pltpu.SemaphoreType.DMA((3,)),
        ],
        compiler_params=pltpu.CompilerParams(collective_id=0),
    )(x, Wq, Wo, Wk, Wv)
